# baseline (speedup 1.0000x reference)
import functools

import jax
import jax.numpy as jnp
from jax import lax
from jax.experimental import pallas as pl
from jax.experimental.pallas import tpu as pltpu
from jax.experimental.pallas import tpu_sc as plsc
import numpy as np

_SIZE = 32000
_PAD = 0
_SMOOTHING = 0.1
_CONFIDENCE = 1.0 - _SMOOTHING
_N_TOK = 4096
_LATENT = 512

_EPS = np.float32(_SMOOTHING / (_SIZE - 2))
_YLOGY = np.float32(
    (_SIZE - 2) * float(_EPS) * np.log(float(_EPS))
    + _CONFIDENCE * np.log(_CONFIDENCE)
)

_RBLK = 4096
_CBLK = 1280
_RGRID = _N_TOK // _RBLK
_CGRID = _SIZE // _CBLK
_ACCW = 128
_NGRP = _CBLK // _ACCW

_SC_NC = 2
_SC_NS = 16
_SC_L = 16
_SC_NW = _SC_NC * _SC_NS
_SC_ROWS = _N_TOK // _SC_NW
_SC_CH = 32


def _loss_body(x_ref, tgt_ref, rec_ref, acc_ref, acc128_ref, tacc128_ref):
    i = pl.program_id(0)
    j = pl.program_id(1)

    @pl.when((i == 0) & (j == 0))
    def _init():
        acc_ref[0] = 0.0
        acc_ref[1] = 0.0
        acc_ref[2] = 0.0
        acc_ref[3] = 0.0
        rec_ref[0, 0] = 0.0

    xb = x_ref[...]
    tgt = tgt_ref[0, 0, :]

    col = j * _CBLK + lax.broadcasted_iota(jnp.int32, (_RBLK, _CBLK), 1)
    hit = col == tgt[:, None]
    rs = xb[:, 0:_ACCW]
    ts = jnp.where(hit[:, 0:_ACCW], xb[:, 0:_ACCW], 0.0)
    for g in range(1, _NGRP):
        sl = slice(g * _ACCW, (g + 1) * _ACCW)
        rs = rs + xb[:, sl]
        ts = ts + jnp.where(hit[:, sl], xb[:, sl], 0.0)

    @pl.when(j == 0)
    def _first_colblock():
        acc128_ref[...] = rs
        tacc128_ref[...] = ts
        w = (tgt != _PAD).astype(jnp.float32)
        acc_ref[1] = acc_ref[1] + jnp.sum(xb[:, _PAD] * w)
        acc_ref[2] = acc_ref[2] + jnp.sum(w)

    @pl.when(j > 0)
    def _accum():
        acc128_ref[...] = acc128_ref[...] + rs
        tacc128_ref[...] = tacc128_ref[...] + ts

    @pl.when(j == _CGRID - 1)
    def _last_colblock():
        w = (tgt != _PAD).astype(jnp.float32)
        rowsum = jnp.sum(acc128_ref[...], axis=1)
        tval = jnp.sum(tacc128_ref[...], axis=1)
        acc_ref[0] = acc_ref[0] + jnp.sum(rowsum * w)
        acc_ref[3] = acc_ref[3] + jnp.sum(tval * w)

    @pl.when((i == _RGRID - 1) & (j == _CGRID - 1))
    def _finalize():
        rec_ref[0, 0] = (
            acc_ref[2] * _YLOGY
            - _EPS * (acc_ref[0] - acc_ref[1])
            - (np.float32(_CONFIDENCE) - _EPS) * acc_ref[3]
        )


_sc_mesh = plsc.VectorSubcoreMesh(core_axis_name="c", subcore_axis_name="s")


@functools.partial(
    pl.kernel,
    mesh=_sc_mesh,
    out_type=jax.ShapeDtypeStruct((_SC_NW, _SC_L), jnp.float32),
    scratch_types=[
        pltpu.VMEM((_SC_CH, _LATENT), jnp.float32),
        pltpu.VMEM((_SC_CH, _LATENT), jnp.float32),
        pltpu.VMEM((_SC_L,), jnp.float32),
    ],
)
def _sc_kl(mu_hbm, lv_hbm, out_hbm, mu_v, lv_v, part_v):
    wid = lax.axis_index("s") * _SC_NC + lax.axis_index("c")
    base = wid * _SC_ROWS

    def chunk_body(c, acc):
        row0 = pl.multiple_of(base + c * _SC_CH, _SC_CH)
        pltpu.sync_copy(mu_hbm.at[pl.ds(row0, _SC_CH)], mu_v)
        pltpu.sync_copy(lv_hbm.at[pl.ds(row0, _SC_CH)], lv_v)

        def row_body(r, acc2):
            for k in range(_LATENT // _SC_L):
                m = mu_v[r, pl.ds(k * _SC_L, _SC_L)]
                l = lv_v[r, pl.ds(k * _SC_L, _SC_L)]
                acc2 = acc2 + (1.0 + l - m * m - jnp.exp(l))
            return acc2

        return lax.fori_loop(0, _SC_CH, row_body, acc)

    acc = lax.fori_loop(0, _SC_ROWS // _SC_CH, chunk_body,
                        jnp.zeros((_SC_L,), jnp.float32))
    part_v[...] = acc
    pltpu.sync_copy(part_v, out_hbm.at[wid])


@jax.jit
def kernel(x, target, mu, logvar):
    tgt3 = target.reshape(_RGRID, 1, _RBLK)
    kl_parts = _sc_kl(mu, logvar)
    (rec,) = pl.pallas_call(
        _loss_body,
        grid=(_RGRID, _CGRID),
        in_specs=[
            pl.BlockSpec((_RBLK, _CBLK), lambda i, j: (i, j)),
            pl.BlockSpec((1, 1, _RBLK), lambda i, j: (i, 0, 0)),
        ],
        out_specs=[
            pl.BlockSpec(memory_space=pltpu.SMEM),
        ],
        out_shape=[
            jax.ShapeDtypeStruct((1, 1), jnp.float32),
        ],
        scratch_shapes=[
            pltpu.SMEM((4,), jnp.float32),
            pltpu.VMEM((_RBLK, _ACCW), jnp.float32),
            pltpu.VMEM((_RBLK, _ACCW), jnp.float32),
        ],
    )(x, tgt3)
    kl = -0.5 * jnp.sum(kl_parts) / np.float32(_N_TOK * _LATENT)
    return (rec[0, 0], kl)


# speedup vs baseline: 1.0060x; 1.0060x over previous
import functools

import jax
import jax.numpy as jnp
from jax import lax
from jax.experimental import pallas as pl
from jax.experimental.pallas import tpu as pltpu
from jax.experimental.pallas import tpu_sc as plsc
import numpy as np

_SIZE = 32000
_PAD = 0
_SMOOTHING = 0.1
_CONFIDENCE = 1.0 - _SMOOTHING
_N_TOK = 4096
_LATENT = 512

_EPS = np.float32(_SMOOTHING / (_SIZE - 2))
_YLOGY = np.float32(
    (_SIZE - 2) * float(_EPS) * np.log(float(_EPS))
    + _CONFIDENCE * np.log(_CONFIDENCE)
)

_RBLK = 2048
_CBLK = 3200
_RGRID = _N_TOK // _RBLK
_CGRID = _SIZE // _CBLK
_ACCW = 128
_NGRP = _CBLK // _ACCW

_SC_NC = 2
_SC_NS = 16
_SC_L = 16
_SC_NW = _SC_NC * _SC_NS
_SC_ROWS = _N_TOK // _SC_NW
_SC_CH = 32


def _loss_body(x_ref, tgt_ref, rec_ref, acc_ref, acc128_ref, tacc128_ref):
    i = pl.program_id(0)
    j = pl.program_id(1)

    @pl.when((i == 0) & (j == 0))
    def _init():
        acc_ref[0] = 0.0
        acc_ref[1] = 0.0
        acc_ref[2] = 0.0
        acc_ref[3] = 0.0
        rec_ref[0, 0] = 0.0

    xb = x_ref[...]
    tgt = tgt_ref[0, 0, :]

    col = j * _CBLK + lax.broadcasted_iota(jnp.int32, (_RBLK, _CBLK), 1)
    hit = col == tgt[:, None]
    rs = xb[:, 0:_ACCW]
    ts = jnp.where(hit[:, 0:_ACCW], xb[:, 0:_ACCW], 0.0)
    for g in range(1, _NGRP):
        sl = slice(g * _ACCW, (g + 1) * _ACCW)
        rs = rs + xb[:, sl]
        ts = ts + jnp.where(hit[:, sl], xb[:, sl], 0.0)

    @pl.when(j == 0)
    def _first_colblock():
        acc128_ref[...] = rs
        tacc128_ref[...] = ts
        w = (tgt != _PAD).astype(jnp.float32)
        acc_ref[1] = acc_ref[1] + jnp.sum(xb[:, _PAD] * w)
        acc_ref[2] = acc_ref[2] + jnp.sum(w)

    @pl.when(j > 0)
    def _accum():
        acc128_ref[...] = acc128_ref[...] + rs
        tacc128_ref[...] = tacc128_ref[...] + ts

    @pl.when(j == _CGRID - 1)
    def _last_colblock():
        w = (tgt != _PAD).astype(jnp.float32)
        rowsum = jnp.sum(acc128_ref[...], axis=1)
        tval = jnp.sum(tacc128_ref[...], axis=1)
        acc_ref[0] = acc_ref[0] + jnp.sum(rowsum * w)
        acc_ref[3] = acc_ref[3] + jnp.sum(tval * w)

    @pl.when((i == _RGRID - 1) & (j == _CGRID - 1))
    def _finalize():
        rec_ref[0, 0] = (
            acc_ref[2] * _YLOGY
            - _EPS * (acc_ref[0] - acc_ref[1])
            - (np.float32(_CONFIDENCE) - _EPS) * acc_ref[3]
        )


_sc_mesh = plsc.VectorSubcoreMesh(core_axis_name="c", subcore_axis_name="s")


@functools.partial(
    pl.kernel,
    mesh=_sc_mesh,
    out_type=jax.ShapeDtypeStruct((_SC_NW, _SC_L), jnp.float32),
    scratch_types=[
        pltpu.VMEM((_SC_CH, _LATENT), jnp.float32),
        pltpu.VMEM((_SC_CH, _LATENT), jnp.float32),
        pltpu.VMEM((_SC_L,), jnp.float32),
    ],
)
def _sc_kl(mu_hbm, lv_hbm, out_hbm, mu_v, lv_v, part_v):
    wid = lax.axis_index("s") * _SC_NC + lax.axis_index("c")
    base = wid * _SC_ROWS

    def chunk_body(c, acc):
        row0 = pl.multiple_of(base + c * _SC_CH, _SC_CH)
        pltpu.sync_copy(mu_hbm.at[pl.ds(row0, _SC_CH)], mu_v)
        pltpu.sync_copy(lv_hbm.at[pl.ds(row0, _SC_CH)], lv_v)

        def row_body(r, acc2):
            for k in range(_LATENT // _SC_L):
                m = mu_v[r, pl.ds(k * _SC_L, _SC_L)]
                l = lv_v[r, pl.ds(k * _SC_L, _SC_L)]
                acc2 = acc2 + (1.0 + l - m * m - jnp.exp(l))
            return acc2

        return lax.fori_loop(0, _SC_CH, row_body, acc)

    acc = lax.fori_loop(0, _SC_ROWS // _SC_CH, chunk_body,
                        jnp.zeros((_SC_L,), jnp.float32))
    part_v[...] = acc
    pltpu.sync_copy(part_v, out_hbm.at[wid])


@jax.jit
def kernel(x, target, mu, logvar):
    tgt3 = target.reshape(_RGRID, 1, _RBLK)
    kl_parts = _sc_kl(mu, logvar)
    (rec,) = pl.pallas_call(
        _loss_body,
        grid=(_RGRID, _CGRID),
        in_specs=[
            pl.BlockSpec((_RBLK, _CBLK), lambda i, j: (i, j)),
            pl.BlockSpec((1, 1, _RBLK), lambda i, j: (i, 0, 0)),
        ],
        out_specs=[
            pl.BlockSpec(memory_space=pltpu.SMEM),
        ],
        out_shape=[
            jax.ShapeDtypeStruct((1, 1), jnp.float32),
        ],
        scratch_shapes=[
            pltpu.SMEM((4,), jnp.float32),
            pltpu.VMEM((_RBLK, _ACCW), jnp.float32),
            pltpu.VMEM((_RBLK, _ACCW), jnp.float32),
        ],
    )(x, tgt3)
    kl = -0.5 * jnp.sum(kl_parts) / np.float32(_N_TOK * _LATENT)
    return (rec[0, 0], kl)
